# baseline (speedup 1.0000x reference)
import jax
import jax.numpy as jnp
from jax.experimental import pallas as pl
from jax.experimental.pallas import tpu as pltpu

_N = 4096
_GRID = 16
_BLK = _N // _GRID
_NCHUNK = 4
_CBLK = _BLK // _NCHUNK
_NSLOT = 2
_DH = 32
_DOUT = 16


def _gcn_body(x_ref, adj_hbm, w1_ref, b1_ref, w2_ref, b2_ref, out_ref,
              buf_ref, a_cache_ref, rhs_ref, sem):
    t = pl.program_id(0)

    def _copy(blk, slot, c):
        return pltpu.make_async_copy(
            adj_hbm.at[pl.ds(blk * _BLK + c * _CBLK, _CBLK), :],
            buf_ref.at[slot, pl.ds(c * _CBLK, _CBLK), :],
            sem.at[slot, c])

    def _start(blk, slot):
        for c in range(_NCHUNK):
            _copy(blk, slot, c).start()

    def _wait(blk, slot):
        for c in range(_NCHUNK):
            _copy(blk, slot, c).wait()

    @pl.when(t == 0)
    def _init():
        _start(0, 0)
        rhs_ref[:, :_DH] = jnp.dot(
            x_ref[...], w1_ref[...],
            preferred_element_type=jnp.float32).astype(jnp.bfloat16)
        rhs_ref[:, _DH:] = jnp.zeros((_N, _DOUT), jnp.bfloat16)

    @pl.when(t < _GRID)
    def _stream():
        @pl.when(t + 1 < _GRID)
        def _prefetch():
            _start(t + 1, (t + 1) % _NSLOT)
        _wait(t, t % _NSLOT)
        a_cache_ref[pl.ds(t * _BLK, _BLK), :] = (
            buf_ref[t % _NSLOT].astype(jnp.bfloat16))
        fused = jnp.dot(a_cache_ref[pl.ds(t * _BLK, _BLK), :], rhs_ref[...],
                        preferred_element_type=jnp.float32)
        h = jnp.maximum(fused[:, :_DH] + b1_ref[...], 0.0)
        g_t = jnp.dot(h.astype(jnp.bfloat16), w2_ref[...],
                      preferred_element_type=jnp.float32).astype(jnp.bfloat16)
        rhs_ref[pl.ds(t * _BLK, _BLK), _DH:] = g_t
        out_ref[pl.ds(t * _BLK, _BLK), :] = fused[:, _DH:] + jnp.dot(
            a_cache_ref[pl.ds(t * _BLK, _BLK), pl.ds(t * _BLK, _BLK)], g_t,
            preferred_element_type=jnp.float32)
        for tt in range(2, _GRID):
            @pl.when(t == tt)
            def _panel(j=tt - 1):
                out_ref[:j * _BLK, :] += jnp.dot(
                    a_cache_ref[:j * _BLK, j * _BLK:(j + 1) * _BLK],
                    rhs_ref[j * _BLK:(j + 1) * _BLK, _DH:],
                    preferred_element_type=jnp.float32)

    @pl.when(t == _GRID)
    def _drain():
        j = _GRID - 1
        out_ref[:j * _BLK, :] += jnp.dot(
            a_cache_ref[:j * _BLK, j * _BLK:(j + 1) * _BLK],
            rhs_ref[j * _BLK:(j + 1) * _BLK, _DH:],
            preferred_element_type=jnp.float32)
        o = (out_ref[...] + b2_ref[...]).reshape(512, 8, 16)
        e = o - jnp.max(o, axis=2, keepdims=True)
        r = e - jnp.log(jnp.sum(jnp.exp(e), axis=2, keepdims=True))
        out_ref[...] = r.reshape(_N, _DOUT)


def kernel(x, adj, W1, b1, W2, b2):
    n, d_in = x.shape
    d_h = W1.shape[1]
    d_out = W2.shape[1]
    b1r = b1.reshape(1, d_h)
    b2r = b2.reshape(1, d_out)

    out = pl.pallas_call(
        _gcn_body,
        grid=(_GRID + 1,),
        in_specs=[
            pl.BlockSpec((n, d_in), lambda t: (0, 0)),
            pl.BlockSpec(memory_space=pl.ANY),
            pl.BlockSpec((d_in, d_h), lambda t: (0, 0)),
            pl.BlockSpec((1, d_h), lambda t: (0, 0)),
            pl.BlockSpec((d_h, d_out), lambda t: (0, 0)),
            pl.BlockSpec((1, d_out), lambda t: (0, 0)),
        ],
        out_specs=pl.BlockSpec((n, d_out), lambda t: (0, 0)),
        out_shape=jax.ShapeDtypeStruct((n, d_out), jnp.float32),
        scratch_shapes=[
            pltpu.VMEM((_NSLOT, _BLK, _N), jnp.float32),
            pltpu.VMEM((_N, _N), jnp.bfloat16),
            pltpu.VMEM((_N, _DH + _DOUT), jnp.bfloat16),
            pltpu.SemaphoreType.DMA((_NSLOT, _NCHUNK)),
        ],
        compiler_params=pltpu.CompilerParams(
            vmem_limit_bytes=100 * 1024 * 1024,
        ),
    )(x, adj, W1, b1r, W2, b2r)
    return out


# speedup vs baseline: 1.1456x; 1.1456x over previous
import jax
import jax.numpy as jnp
from jax.experimental import pallas as pl
from jax.experimental.pallas import tpu as pltpu

_N = 4096
_GRID = 8
_BLK = _N // _GRID
_NCHUNK = 8
_CBLK = _BLK // _NCHUNK
_NSLOT = 2
_DH = 32
_DOUT = 16


def _gcn_body(x_ref, adj_hbm, w1_ref, b1_ref, w2_ref, b2_ref, out_ref,
              buf_ref, a_cache_ref, rhs_ref, sem):
    t = pl.program_id(0)

    def _copy(blk, slot, c):
        return pltpu.make_async_copy(
            adj_hbm.at[pl.ds(blk * _BLK + c * _CBLK, _CBLK), :],
            buf_ref.at[slot, pl.ds(c * _CBLK, _CBLK), :],
            sem.at[slot, c])

    def _start(blk, slot):
        for c in range(_NCHUNK):
            _copy(blk, slot, c).start()

    def _wait(blk, slot):
        for c in range(_NCHUNK):
            _copy(blk, slot, c).wait()

    @pl.when(t == 0)
    def _init():
        _start(0, 0)
        rhs_ref[:, :_DH] = jnp.dot(
            x_ref[...], w1_ref[...],
            preferred_element_type=jnp.float32).astype(jnp.bfloat16)
        rhs_ref[:, _DH:] = jnp.zeros((_N, _DOUT), jnp.bfloat16)

    @pl.when(t < _GRID)
    def _stream():
        @pl.when(t + 1 < _GRID)
        def _prefetch():
            _start(t + 1, (t + 1) % _NSLOT)
        _wait(t, t % _NSLOT)
        a_cache_ref[pl.ds(t * _BLK, _BLK), :] = (
            buf_ref[t % _NSLOT].astype(jnp.bfloat16))
        fused = jnp.dot(a_cache_ref[pl.ds(t * _BLK, _BLK), :], rhs_ref[...],
                        preferred_element_type=jnp.float32)
        h = jnp.maximum(fused[:, :_DH] + b1_ref[...], 0.0)
        g_t = jnp.dot(h.astype(jnp.bfloat16), w2_ref[...],
                      preferred_element_type=jnp.float32).astype(jnp.bfloat16)
        rhs_ref[pl.ds(t * _BLK, _BLK), _DH:] = g_t
        out_ref[pl.ds(t * _BLK, _BLK), :] = fused[:, _DH:] + jnp.dot(
            a_cache_ref[pl.ds(t * _BLK, _BLK), pl.ds(t * _BLK, _BLK)], g_t,
            preferred_element_type=jnp.float32)
        for tt in range(2, _GRID):
            @pl.when(t == tt)
            def _panel(j=tt - 1):
                out_ref[:j * _BLK, :] += jnp.dot(
                    a_cache_ref[:j * _BLK, j * _BLK:(j + 1) * _BLK],
                    rhs_ref[j * _BLK:(j + 1) * _BLK, _DH:],
                    preferred_element_type=jnp.float32)

    @pl.when(t == _GRID)
    def _drain():
        j = _GRID - 1
        out_ref[:j * _BLK, :] += jnp.dot(
            a_cache_ref[:j * _BLK, j * _BLK:(j + 1) * _BLK],
            rhs_ref[j * _BLK:(j + 1) * _BLK, _DH:],
            preferred_element_type=jnp.float32)
        o = (out_ref[...] + b2_ref[...]).reshape(512, 8, 16)
        e = o - jnp.max(o, axis=2, keepdims=True)
        r = e - jnp.log(jnp.sum(jnp.exp(e), axis=2, keepdims=True))
        out_ref[...] = r.reshape(_N, _DOUT)


def kernel(x, adj, W1, b1, W2, b2):
    n, d_in = x.shape
    d_h = W1.shape[1]
    d_out = W2.shape[1]
    b1r = b1.reshape(1, d_h)
    b2r = b2.reshape(1, d_out)

    out = pl.pallas_call(
        _gcn_body,
        grid=(_GRID + 1,),
        in_specs=[
            pl.BlockSpec((n, d_in), lambda t: (0, 0)),
            pl.BlockSpec(memory_space=pl.ANY),
            pl.BlockSpec((d_in, d_h), lambda t: (0, 0)),
            pl.BlockSpec((1, d_h), lambda t: (0, 0)),
            pl.BlockSpec((d_h, d_out), lambda t: (0, 0)),
            pl.BlockSpec((1, d_out), lambda t: (0, 0)),
        ],
        out_specs=pl.BlockSpec((n, d_out), lambda t: (0, 0)),
        out_shape=jax.ShapeDtypeStruct((n, d_out), jnp.float32),
        scratch_shapes=[
            pltpu.VMEM((_NSLOT, _BLK, _N), jnp.float32),
            pltpu.VMEM((_N, _N), jnp.bfloat16),
            pltpu.VMEM((_N, _DH + _DOUT), jnp.bfloat16),
            pltpu.SemaphoreType.DMA((_NSLOT, _NCHUNK)),
        ],
        compiler_params=pltpu.CompilerParams(
            vmem_limit_bytes=100 * 1024 * 1024,
        ),
    )(x, adj, W1, b1r, W2, b2r)
    return out
